# baseline (speedup 1.0000x reference)
import functools

import jax
import jax.numpy as jnp
from jax import lax
from jax.experimental import pallas as pl
from jax.experimental.pallas import tpu as pltpu
from jax.experimental.pallas import tpu_sc as plsc

EMB = 64
B_USERS = 16384
N_NEG = 65536
NEG_PER_USER = N_NEG // B_USERS
NUM_WORKERS = 32
USERS_PER_WORKER = B_USERS // NUM_WORKERS
SUB = 128
NSUB = USERS_PER_WORKER // SUB
IDX_LANES = 128


def _sc_body(uidx_hbm, pidx_hbm, nidx_hbm, uemb_hbm, iemb_hbm,
             pos_out, neg_out,
             uidx_v, pidx_v, nidx_v, U, P, N, pos_v, neg_v, sem):
    w = lax.axis_index("s") * 2 + lax.axis_index("c")
    iota = lax.broadcasted_iota(jnp.int32, (16,), 0)

    pltpu.sync_copy(uidx_hbm.at[pl.ds(w * NSUB, NSUB)], uidx_v)
    pltpu.sync_copy(pidx_hbm.at[pl.ds(w * NSUB, NSUB)], pidx_v)
    pltpu.sync_copy(nidx_hbm.at[pl.ds(w * (NSUB * 4), NSUB * 4)], nidx_v)

    for c in range(NSUB):
        cps = [
            pltpu.async_copy(uemb_hbm.at[uidx_v.at[c]], U, sem),
            pltpu.async_copy(iemb_hbm.at[pidx_v.at[c]], P, sem),
        ]
        for j in range(4):
            cps.append(pltpu.async_copy(
                iemb_hbm.at[nidx_v.at[4 * c + j]],
                N.at[pl.ds(IDX_LANES * j, IDX_LANES)], sem))
        for cp in cps:
            cp.wait()

        def gbody(g, _):
            urow = iota + g * 16
            nrows = [4 * iota + g * 64 + n for n in range(NEG_PER_USER)]
            col = lambda k: jnp.full((16,), k, jnp.int32)
            u0 = plsc.load_gather(U, [urow, col(0)])
            acc_p = u0 + plsc.load_gather(P, [urow, col(63)])
            accs = [u0 + plsc.load_gather(N, [nr, col(63)]) for nr in nrows]
            for k in range(EMB - 1):
                ucol = plsc.load_gather(U, [urow, col(k + 1)])
                acc_p = acc_p + ucol * plsc.load_gather(P, [urow, col(k)])
                for n in range(NEG_PER_USER):
                    accs[n] = accs[n] + ucol * plsc.load_gather(
                        N, [nrows[n], col(k)])
            pos_v[pl.ds(g * 16, 16)] = 1.0 / (1.0 + jnp.exp(-acc_p))
            for n in range(NEG_PER_USER):
                plsc.store_scatter(neg_v, [4 * iota + g * 64 + n],
                                   1.0 / (1.0 + jnp.exp(-accs[n])))
            return 0

        lax.fori_loop(0, SUB // 16, gbody, 0)

        pltpu.sync_copy(pos_v, pos_out.at[pl.ds(w * USERS_PER_WORKER + c * SUB, SUB)])
        pltpu.sync_copy(neg_v, neg_out.at[pl.ds((w * NSUB + c) * (SUB * 4), SUB * 4)])


_sc_kernel = functools.partial(
    pl.kernel,
    mesh=plsc.VectorSubcoreMesh(core_axis_name="c", subcore_axis_name="s"),
    compiler_params=pltpu.CompilerParams(
        needs_layout_passes=False, use_tc_tiling_on_sc=False),
    out_type=(
        jax.ShapeDtypeStruct((B_USERS,), jnp.float32),
        jax.ShapeDtypeStruct((N_NEG,), jnp.float32),
    ),
    scratch_types=[
        pltpu.VMEM((NSUB, IDX_LANES), jnp.int32),
        pltpu.VMEM((NSUB, IDX_LANES), jnp.int32),
        pltpu.VMEM((NSUB * 4, IDX_LANES), jnp.int32),
        pltpu.VMEM((SUB, EMB), jnp.float32),
        pltpu.VMEM((SUB, EMB), jnp.float32),
        pltpu.VMEM((SUB * 4, EMB), jnp.float32),
        pltpu.VMEM((SUB,), jnp.float32),
        pltpu.VMEM((SUB * 4,), jnp.float32),
        pltpu.SemaphoreType.DMA,
    ],
)(_sc_body)


@jax.jit
def kernel(batch_pos, neg_item_index, user_embedding, item_embedding):
    user_num = user_embedding.shape[0]
    uidx = batch_pos[:, 0].astype(jnp.int32).reshape(-1, IDX_LANES)
    pidx = (batch_pos[:, 1] - user_num).astype(jnp.int32).reshape(-1, IDX_LANES)
    nidx = (neg_item_index - user_num).astype(jnp.int32).reshape(-1, IDX_LANES)
    pos, neg = _sc_kernel(uidx, pidx, nidx, user_embedding, item_embedding)
    return pos.reshape(-1, 1), neg.reshape(-1, 1)
